# Initial kernel scaffold; baseline (speedup 1.0000x reference)
import jax
import jax.numpy as jnp
from jax.experimental import pallas as pl


def kernel(im, noise, fill_noise):
    raise NotImplementedError("write your pallas kernel here")


# speedup vs baseline: 4.5248x; 4.5248x over previous
import functools

import jax
import jax.numpy as jnp
from jax import lax
from jax.experimental import pallas as pl
from jax.experimental.pallas import tpu as pltpu
from jax.experimental.pallas import tpu_sc as plsc

_P = 16
_L = 576
_KEEP = 144
_NFILL = 432
_G = 24
_N = 64
_C = 3


def _rank_body(noise_ref, ids_ref, mask_ref):
    a = noise_ref[0, 0, :]
    col = a[:, None]
    row = a[None, :]
    il = lax.broadcasted_iota(jnp.int32, (_L, _L), 0)
    im_ = lax.broadcasted_iota(jnp.int32, (_L, _L), 1)
    cond = (row < col) | ((row == col) & (im_ < il))
    rank = jnp.sum(cond.astype(jnp.int32), axis=1)
    ids_ref[0, 0, :] = rank
    mask_ref[0, 0, :] = (rank >= _KEEP).astype(jnp.float32)


def _compute_ranks(noise):
    noise3 = noise.reshape(_N, 1, _L)
    ids3, mask3 = pl.pallas_call(
        _rank_body,
        grid=(_N,),
        in_specs=[pl.BlockSpec((1, 1, _L), lambda i: (i, 0, 0))],
        out_specs=[
            pl.BlockSpec((1, 1, _L), lambda i: (i, 0, 0)),
            pl.BlockSpec((1, 1, _L), lambda i: (i, 0, 0)),
        ],
        out_shape=[
            jax.ShapeDtypeStruct((_N, 1, _L), jnp.int32),
            jax.ShapeDtypeStruct((_N, 1, _L), jnp.float32),
        ],
    )(noise3)
    return ids3.reshape(_N, _L), mask3.reshape(_N, _L)


_NBLOCKS = _N * _G
_PER_W = _NBLOCKS // 32


@functools.cache
def _build_sc_masker():
    return functools.partial(
        pl.kernel,
        mesh=plsc.VectorSubcoreMesh(core_axis_name="c", subcore_axis_name="s"),
        compiler_params=pltpu.CompilerParams(needs_layout_passes=False),
        out_type=jax.ShapeDtypeStruct((_N, _C, 384, 384), jnp.float32),
        scratch_types=[
            pltpu.VMEM((_C, _P, 384), jnp.float32),
            pltpu.VMEM((_G, 768), jnp.float32),
            pltpu.VMEM((32,), jnp.int32),
            pltpu.VMEM((_G,), jnp.int32),
            pltpu.SemaphoreType.DMA,
        ],
    )(_sc_body)


def _sc_body(im_hbm, fill_hbm, rank_hbm, out_hbm,
             imbuf, fbuf, rkbuf, idxbuf, sem):
    wid = lax.axis_index("s") * 2 + lax.axis_index("c")
    lane = lax.iota(jnp.int32, 16)

    def block_body(i, carry):
        b = wid * _PER_W + i
        n = b // _G
        hh = b % _G
        for c in range(_C):
            pltpu.sync_copy(im_hbm.at[n, c, pl.ds(hh * _P, _P), :],
                            imbuf.at[c])
        pltpu.sync_copy(rank_hbm.at[pl.ds(n * 640 + hh * _G, 32)], rkbuf)
        r0 = rkbuf[pl.ds(0, 16)]
        r1 = rkbuf[pl.ds(8, 16)]
        base = n * _NFILL
        idxbuf[pl.ds(0, 16)] = jnp.maximum(r0 - _KEEP, 0) + base
        idxbuf[pl.ds(8, 16)] = jnp.maximum(r1 - _KEEP, 0) + base
        pltpu.async_copy(fill_hbm.at[idxbuf], fbuf, sem).wait()

        def patch_body(j, carry2):
            jv = jnp.full((16,), j, jnp.int32)
            rkv = plsc.load_gather(rkbuf, [jv])
            msk = rkv >= _KEEP
            colidx = lane + j * _P
            for c in range(_C):
                cv = jnp.full((16,), c, jnp.int32)
                for p in range(_P):
                    src = plsc.load_gather(
                        fbuf, [jv, lane * 3 + (p * 48 + c)])
                    plsc.store_scatter(
                        imbuf,
                        [cv, jnp.full((16,), p, jnp.int32), colidx],
                        src, mask=msk)
            return carry2

        lax.fori_loop(0, _G, patch_body, 0)
        for c in range(_C):
            pltpu.sync_copy(imbuf.at[c],
                            out_hbm.at[n, c, pl.ds(hh * _P, _P), :])
        return carry

    lax.fori_loop(0, _PER_W, block_body, 0)


def kernel(im, noise, fill_noise):
    ids_restore, mask = _compute_ranks(noise)
    rank_pad = jnp.pad(ids_restore, ((0, 0), (0, 64))).reshape(-1)
    fill_flat = fill_noise.reshape(_N * _NFILL, 768)
    masked_img = _build_sc_masker()(im, fill_flat, rank_pad)
    return masked_img, mask, ids_restore
